# Initial kernel scaffold; baseline (speedup 1.0000x reference)
import jax
import jax.numpy as jnp
from jax.experimental import pallas as pl


def kernel(node_states, edges, indices, W1, a1, W2, a2):
    raise NotImplementedError("write your pallas kernel here")


# speedup vs baseline: 43.5198x; 43.5198x over previous
import functools

import jax
import jax.numpy as jnp
import numpy as np
from jax import lax
from jax.experimental import pallas as pl
from jax.experimental.pallas import tpu as pltpu
from jax.experimental.pallas import tpu_sc as plsc

F32 = jnp.float32
NC, NS, LANES = 2, 16, 16
NW = NC * NS
EC = 80
W = 128

_MESH = plsc.VectorSubcoreMesh(
    core_axis_name="c", subcore_axis_name="s", num_cores=NC, num_subcores=NS
)

_GDN = lax.GatherDimensionNumbers(
    offset_dims=(), collapsed_slice_dims=(0,), start_index_map=(0,)
)


def _vgather(v, idx):
    return lax.gather(v, idx[:, None], _GDN, (1,),
                      mode=lax.GatherScatterMode.PROMISE_IN_BOUNDS)


def _prep1_body(x_ref, w_ref, msa_ref, mtb_ref, mhb_ref, a_ref, b_ref):
    h = jnp.dot(x_ref[...], w_ref[...], preferred_element_type=F32)
    lr = jnp.where(h > 0, h, 0.2 * h)
    a_ref[...] = jnp.dot(lr, msa_ref[...], preferred_element_type=F32)
    b_ref[...] = (jnp.dot(lr, mtb_ref[...], preferred_element_type=F32)
                  + jnp.dot(h, mhb_ref[...], preferred_element_type=F32))


def _prep2_body(p0_ref, p1_ref, md_ref, mn_ref, w2_ref, msa2_ref, mtb2_ref,
                mhb2_ref, a2_ref, b2_ref):
    acc = p0_ref[...] + p1_ref[...]
    den = jnp.dot(acc, md_ref[...], preferred_element_type=F32)
    num = jnp.dot(acc, mn_ref[...], preferred_element_type=F32)
    x2 = jnp.maximum(num / (den + 1e-9), 0.0)
    h2 = jnp.dot(x2, w2_ref[...], preferred_element_type=F32)
    lr2 = jnp.where(h2 > 0, h2, 0.2 * h2)
    a2_ref[...] = jnp.dot(lr2, msa2_ref[...], preferred_element_type=F32)
    b2_ref[...] = (jnp.dot(lr2, mtb2_ref[...], preferred_element_type=F32)
                   + jnp.dot(h2, mhb2_ref[...], preferred_element_type=F32))


def _edge_body(src_hbm, tgt_hbm, a_hbm, b_hbm, out_hbm,
               acc_sh, srcv, tgtv, sbuf, tbuf, contrib, sem_a, sem_b,
               *, layer):
    core = lax.axis_index("c")
    sub = lax.axis_index("s")
    wid = sub * NC + core
    n_pad = acc_sh.shape[0]
    nper = src_hbm.shape[0] // NW
    nblocks = nper // EC
    rps = n_pad // NS
    nchunks = rps // EC
    zeros16 = jnp.zeros((LANES,), F32)
    ii = lax.iota(jnp.int32, LANES)
    rs = sub * rps

    def _zrow(i, c):
        for o in range(0, W, LANES):
            contrib[i, pl.ds(o, LANES)] = zeros16
        return c
    lax.fori_loop(0, EC, _zrow, 0)
    for k in range(nchunks):
        pltpu.sync_copy(contrib, acc_sh.at[pl.ds(rs + k * EC, EC)])
    plsc.subcore_barrier()

    bidx = [(ii >> 3) + 2 * c for c in range(4)]
    zidx = ii * 0

    def _block(b, c):
        base = wid * nper + b * EC
        pltpu.sync_copy(src_hbm.at[pl.ds(base, EC)], srcv)
        pltpu.sync_copy(tgt_hbm.at[pl.ds(base, EC)], tgtv)
        cpa = pltpu.async_copy(a_hbm.at[srcv], sbuf, sem_a)
        cpb = pltpu.async_copy(b_hbm.at[tgtv], tbuf, sem_b)
        cpa.wait()
        cpb.wait()

        if layer == 1:
            def _edge(i, c2):
                sv = sbuf[i, pl.ds(0, LANES)]
                tv = tbuf[i, pl.ds(0, LANES)]
                ev = jnp.exp(jnp.clip(sv + tv, -2.0, 2.0))
                contrib[i, pl.ds(0, LANES)] = ev
                for v in range(4):
                    hc = tbuf[i, pl.ds(8 + LANES * v, LANES)]
                    eb = _vgather(ev, bidx[v])
                    contrib[i, pl.ds(8 + LANES * v, LANES)] = eb * hc
                return c2
        else:
            def _edge(i, c2):
                sv = sbuf[i, pl.ds(0, LANES)]
                tv = tbuf[i, pl.ds(0, LANES)]
                sm = sv + tv
                ev = jnp.exp(jnp.clip(sm, -2.0, 2.0))
                e2 = _vgather(ev, zidx)
                contrib[i, pl.ds(0, LANES)] = e2 * jnp.where(ii == 0, 1.0, sm)
                return c2
        lax.fori_loop(0, EC, _edge, 0)
        pltpu.sync_copy(contrib, acc_sh.at[srcv], add=True)
        return c
    lax.fori_loop(0, nblocks, _block, 0)

    plsc.subcore_barrier()
    for k in range(nchunks):
        pltpu.sync_copy(acc_sh.at[pl.ds(rs + k * EC, EC)], contrib)
        pltpu.sync_copy(contrib, out_hbm.at[core, pl.ds(rs + k * EC, EC)])


def _fin_body(q0_hbm, q1_hbm, idx_hbm, out_hbm, idxv, r0, r1, sem0, sem1):
    core = lax.axis_index("c")
    sub = lax.axis_index("s")
    wid = sub * NC + core
    per = idx_hbm.shape[0] // NW
    nb = per // EC
    ii = lax.iota(jnp.int32, LANES)
    zidx = ii * 0

    def _block(b, c):
        base = wid * per + b * EC
        pltpu.sync_copy(idx_hbm.at[pl.ds(base, EC)], idxv)
        cp0 = pltpu.async_copy(q0_hbm.at[idxv], r0, sem0)
        cp1 = pltpu.async_copy(q1_hbm.at[idxv], r1, sem1)
        cp0.wait()
        cp1.wait()

        def _row(i, c2):
            srow = r0[i, pl.ds(0, LANES)] + r1[i, pl.ds(0, LANES)]
            den = _vgather(srow, zidx)
            r0[i, pl.ds(0, LANES)] = jnp.maximum(srow / (den + 1e-9), 0.0)
            return c2
        lax.fori_loop(0, EC, _row, 0)
        pltpu.sync_copy(r0, out_hbm.at[pl.ds(base, EC)])
        return c
    lax.fori_loop(0, nb, _block, 0)


def _make_edge_call(n_pad, layer):
    body = functools.partial(_edge_body, layer=layer)
    return pl.kernel(
        body,
        out_type=jax.ShapeDtypeStruct((NC, n_pad, W), F32),
        mesh=_MESH,
        scratch_types=[
            pltpu.VMEM_SHARED((n_pad, W), F32),
            pltpu.VMEM((EC,), jnp.int32),
            pltpu.VMEM((EC,), jnp.int32),
            pltpu.VMEM((EC, W), F32),
            pltpu.VMEM((EC, W), F32),
            pltpu.VMEM((EC, W), F32),
            pltpu.SemaphoreType.DMA,
            pltpu.SemaphoreType.DMA,
        ],
    )


def kernel(node_states, edges, indices, W1, a1, W2, a2):
    n, d = node_states.shape
    h1, _, u1 = W1.shape
    hu = h1 * u1
    od = W2.shape[2]
    nidx = indices.shape[0]

    w1f = jnp.transpose(W1, (1, 0, 2)).reshape(d, hu)
    rows = np.arange(hu)
    heads = rows // u1
    msa = jnp.zeros((hu, W), F32).at[rows, heads].set(a1[:, :u1, 0].reshape(hu))
    mtb = jnp.zeros((hu, W), F32).at[rows, heads].set(a1[:, u1:, 0].reshape(hu))
    mhb = np.zeros((hu, W), np.float32)
    mhb[rows, rows + 8] = 1.0
    mhb = jnp.asarray(mhb)
    md = np.zeros((W, hu), np.float32)
    md[heads, rows] = 1.0
    md = jnp.asarray(md)
    mn = np.zeros((W, hu), np.float32)
    mn[rows + 8, rows] = 1.0
    mn = jnp.asarray(mn)
    w2p = jnp.concatenate([W2[0], jnp.zeros((hu, 8 - od), F32)], axis=1)
    j7 = np.arange(od)
    msa2 = jnp.zeros((8, W), F32).at[j7, 0].set(a2[0, :od, 0])
    mtb2 = jnp.zeros((8, W), F32).at[j7, 0].set(a2[0, od:, 0])
    mhb2 = np.zeros((8, W), np.float32)
    mhb2[j7, j7 + 1] = 1.0
    mhb2 = jnp.asarray(mhb2)

    src = edges[:, 0]
    tgt = edges[:, 1]

    npd = ((n + NS * EC - 1) // (NS * EC)) * (NS * EC)
    xp = jnp.concatenate([node_states, jnp.zeros((npd - n, d), F32)], axis=0)
    blk = 2048
    grid = (npd // blk,)
    full = lambda i: (0, 0)
    rowb = lambda i: (i, 0)
    prep1 = pl.pallas_call(
        _prep1_body,
        grid=grid,
        in_specs=[
            pl.BlockSpec((blk, d), rowb),
            pl.BlockSpec((d, hu), full),
            pl.BlockSpec((hu, W), full),
            pl.BlockSpec((hu, W), full),
            pl.BlockSpec((hu, W), full),
        ],
        out_specs=[pl.BlockSpec((blk, W), rowb), pl.BlockSpec((blk, W), rowb)],
        out_shape=[jax.ShapeDtypeStruct((npd, W), F32),
                   jax.ShapeDtypeStruct((npd, W), F32)],
    )
    tab_a1, tab_b1 = prep1(xp, w1f, msa, mtb, mhb)

    edge1 = _make_edge_call(npd, 1)
    p1 = edge1(src, tgt, tab_a1, tab_b1)

    prep2 = pl.pallas_call(
        _prep2_body,
        grid=grid,
        in_specs=[
            pl.BlockSpec((blk, W), rowb),
            pl.BlockSpec((blk, W), rowb),
            pl.BlockSpec((W, hu), full),
            pl.BlockSpec((W, hu), full),
            pl.BlockSpec((hu, 8), full),
            pl.BlockSpec((8, W), full),
            pl.BlockSpec((8, W), full),
            pl.BlockSpec((8, W), full),
        ],
        out_specs=[pl.BlockSpec((blk, W), rowb), pl.BlockSpec((blk, W), rowb)],
        out_shape=[jax.ShapeDtypeStruct((npd, W), F32),
                   jax.ShapeDtypeStruct((npd, W), F32)],
    )
    tab_a2, tab_b2 = prep2(p1[0], p1[1], md, mn, w2p, msa2, mtb2, mhb2)

    edge2 = _make_edge_call(npd, 2)
    q = edge2(src, tgt, tab_a2, tab_b2)

    npad = ((nidx + NW * EC - 1) // (NW * EC)) * (NW * EC)
    idxp = jnp.concatenate([indices, jnp.zeros((npad - nidx,), jnp.int32)])
    fin = pl.kernel(
        _fin_body,
        out_type=jax.ShapeDtypeStruct((npad, W), F32),
        mesh=_MESH,
        scratch_types=[
            pltpu.VMEM((EC,), jnp.int32),
            pltpu.VMEM((EC, W), F32),
            pltpu.VMEM((EC, W), F32),
            pltpu.SemaphoreType.DMA,
            pltpu.SemaphoreType.DMA,
        ],
    )
    o = fin(q[0], q[1], idxp)
    return o[:nidx, 1:1 + od]
